# baseline (speedup 1.0000x reference)
import functools

import jax
import jax.numpy as jnp
from jax.experimental import pallas as pl

N = 10000
F_IN = 256
H = 128
C = 64
K = 10
ALPHA = 0.1

ROW_BLK = 1000


def _mlp_body(x_ref, w1_ref, b1_ref, w2_ref, b2_ref, o_ref):
    h = jnp.maximum(
        jnp.dot(x_ref[...], w1_ref[...], preferred_element_type=jnp.float32)
        + b1_ref[...],
        0.0,
    )
    o_ref[...] = (
        jnp.dot(h, w2_ref[...], preferred_element_type=jnp.float32) + b2_ref[...]
    )


@jax.jit
def _mlp(x, W1, b1, W2, b2):
    grid = (N // ROW_BLK,)
    return pl.pallas_call(
        _mlp_body,
        grid=grid,
        in_specs=[
            pl.BlockSpec((ROW_BLK, F_IN), lambda i: (i, 0)),
            pl.BlockSpec((F_IN, H), lambda i: (0, 0)),
            pl.BlockSpec((1, H), lambda i: (0, 0)),
            pl.BlockSpec((H, C), lambda i: (0, 0)),
            pl.BlockSpec((1, C), lambda i: (0, 0)),
        ],
        out_specs=pl.BlockSpec((ROW_BLK, C), lambda i: (i, 0)),
        out_shape=jax.ShapeDtypeStruct((N, C), jnp.float32),
    )(x, W1, b1.reshape(1, H), W2, b2.reshape(1, C))


def kernel(x, edge_index, W1, b1, W2, b2):
    h = _mlp(x, W1, b1, W2, b2)
    n = N
    loop = jnp.arange(n, dtype=edge_index.dtype)
    src = jnp.concatenate([edge_index[0], loop])
    dst = jnp.concatenate([edge_index[1], loop])
    ones = jnp.ones(src.shape[0], dtype=h.dtype)
    deg = jax.ops.segment_sum(ones, dst, num_segments=n)
    dinv = jnp.where(deg > 0, 1.0 / jnp.sqrt(deg), 0.0)
    norm = dinv[src] * dinv[dst]
    z = h
    for _ in range(K):
        msg = norm[:, None] * z[src]
        agg = jax.ops.segment_sum(msg, dst, num_segments=n)
        z = (1.0 - ALPHA) * agg + ALPHA * h
    out = jax.nn.log_softmax(z, axis=1)
    return (out, z)


# speedup vs baseline: 2.8866x; 2.8866x over previous
import jax
import jax.numpy as jnp
from jax import lax
from jax.experimental import pallas as pl
from jax.experimental.pallas import tpu as pltpu
from jax.experimental.pallas import tpu_sc as plsc

N = 10000
F_IN = 256
H = 128
C = 64
K = 10
ALPHA = 0.1

NC, NS = 2, 16
N_PAD = 10240
HALF = N_PAD // NC
A_ROWS = 5248
RPT = A_ROWS // NS
DROW = HALF
WROWS = 10496
CW = 128
E = 160000
CH = 128
NCH_TILE = 80
EPT = CH * NCH_TILE
E_PAD = EPT * NS
IDX_WIN = 8
ECHUNKS = NCH_TILE // IDX_WIN

ROW_BLK = 1024
NBLK = HALF // ROW_BLK
MLP_BLK = 1000


def _mlp_body(x_ref, w1_ref, b1_ref, w2_ref, b2_ref, o_ref):
    hid = jnp.maximum(
        jnp.dot(x_ref[...], w1_ref[...], preferred_element_type=jnp.float32)
        + b1_ref[...],
        0.0,
    )
    o_ref[...] = (
        jnp.dot(hid, w2_ref[...], preferred_element_type=jnp.float32) + b2_ref[...]
    )


def _mlp(x, W1, b1, W2, b2):
    return pl.pallas_call(
        _mlp_body,
        grid=(N // MLP_BLK,),
        in_specs=[
            pl.BlockSpec((MLP_BLK, F_IN), lambda i: (i, 0)),
            pl.BlockSpec((F_IN, H), lambda i: (0, 0)),
            pl.BlockSpec((1, H), lambda i: (0, 0)),
            pl.BlockSpec((H, C), lambda i: (0, 0)),
            pl.BlockSpec((1, C), lambda i: (0, 0)),
        ],
        out_specs=pl.BlockSpec((MLP_BLK, C), lambda i: (i, 0)),
        out_shape=jax.ShapeDtypeStruct((N, C), jnp.float32),
    )(x, W1, b1.reshape(1, H), W2, b2.reshape(1, C))


_INIT_CHUNKS = ((0, 128), (128, 128), (256, 72))


def _prop_body(w_hbm, srcT_hbm, dstT_hbm, t_hbm, sidx, didx, rowbuf, sem, acc):
    c = lax.axis_index("c")
    s = lax.axis_index("s")
    base = s * RPT
    for off, rows in _INIT_CHUNKS:
        pltpu.sync_copy(
            w_hbm.at[pl.ds(c * HALF + base + off, rows)], rowbuf.at[pl.ds(0, rows)]
        )
        pltpu.sync_copy(rowbuf.at[pl.ds(0, rows)], acc.at[pl.ds(base + off, rows)])
    plsc.subcore_barrier()

    for jo in range(ECHUNKS):
        pltpu.sync_copy(srcT_hbm.at[pl.ds(s * NCH_TILE + jo * IDX_WIN, IDX_WIN)], sidx)
        pltpu.sync_copy(
            dstT_hbm.at[pl.ds((c * NS + s) * NCH_TILE + jo * IDX_WIN, IDX_WIN)], didx
        )

        @pl.loop(0, IDX_WIN)
        def inner(ji):
            pltpu.async_copy(w_hbm.at[sidx.at[ji]], rowbuf, sem).wait()
            pltpu.sync_copy(rowbuf, acc.at[didx.at[ji]], add=True)

    plsc.subcore_barrier()
    for off, rows in _INIT_CHUNKS:
        pltpu.sync_copy(acc.at[pl.ds(base + off, rows)], rowbuf.at[pl.ds(0, rows)])
        pltpu.sync_copy(rowbuf.at[pl.ds(0, rows)], t_hbm.at[c, pl.ds(base + off, rows)])


_prop = pl.kernel(
    _prop_body,
    out_type=jax.ShapeDtypeStruct((NC, A_ROWS, CW), jnp.float32),
    mesh=plsc.VectorSubcoreMesh(core_axis_name="c", subcore_axis_name="s"),
    scratch_types=[
        pltpu.VMEM((IDX_WIN, CH), jnp.int32),
        pltpu.VMEM((IDX_WIN, CH), jnp.int32),
        pltpu.VMEM((CH, CW), jnp.float32),
        pltpu.SemaphoreType.DMA,
        pltpu.VMEM_SHARED((A_ROWS, CW), jnp.float32),
    ],
)


def _prep_body(t_ref, h_ref, w2_ref, g_ref, d2_ref, di_ref):
    tt = t_ref[0]
    deg = tt[:, 0:1]
    dinv = jnp.where(deg > 0.0, lax.rsqrt(deg), 0.0)
    w0 = dinv * h_ref[...]
    w2_ref[...] = jnp.pad(w0, ((0, 0), (0, CW - C)))
    g_ref[...] = ALPHA * w0
    d2_ref[...] = jnp.broadcast_to(dinv * dinv, (ROW_BLK, C))
    di_ref[...] = jnp.broadcast_to(dinv, (ROW_BLK, C))


def _prep(t, h_pad):
    return pl.pallas_call(
        _prep_body,
        grid=(NC, NBLK),
        in_specs=[
            pl.BlockSpec((1, ROW_BLK, CW), lambda c, i: (c, i, 0)),
            pl.BlockSpec((ROW_BLK, C), lambda c, i: (c * NBLK + i, 0)),
        ],
        out_specs=[
            pl.BlockSpec((ROW_BLK, CW), lambda c, i: (c * NBLK + i, 0)),
            pl.BlockSpec((ROW_BLK, C), lambda c, i: (c * NBLK + i, 0)),
            pl.BlockSpec((ROW_BLK, C), lambda c, i: (c * NBLK + i, 0)),
            pl.BlockSpec((ROW_BLK, C), lambda c, i: (c * NBLK + i, 0)),
        ],
        out_shape=[
            jax.ShapeDtypeStruct((WROWS, CW), jnp.float32),
            jax.ShapeDtypeStruct((N_PAD, C), jnp.float32),
            jax.ShapeDtypeStruct((N_PAD, C), jnp.float32),
            jax.ShapeDtypeStruct((N_PAD, C), jnp.float32),
        ],
    )(t, h_pad)


def _combine_body(t_ref, d2_ref, g_ref, o_ref):
    tt = t_ref[0][:, :C]
    wn = (1.0 - ALPHA) * d2_ref[...] * tt + g_ref[...]
    o_ref[...] = jnp.pad(wn, ((0, 0), (0, CW - C)))


def _combine(t, d2, g):
    return pl.pallas_call(
        _combine_body,
        grid=(NC, NBLK),
        in_specs=[
            pl.BlockSpec((1, ROW_BLK, CW), lambda c, i: (c, i, 0)),
            pl.BlockSpec((ROW_BLK, C), lambda c, i: (c * NBLK + i, 0)),
            pl.BlockSpec((ROW_BLK, C), lambda c, i: (c * NBLK + i, 0)),
        ],
        out_specs=pl.BlockSpec((ROW_BLK, CW), lambda c, i: (c * NBLK + i, 0)),
        out_shape=jax.ShapeDtypeStruct((WROWS, CW), jnp.float32),
    )(t, d2, g)


def _final_body(t_ref, di_ref, h_ref, out_ref, z_ref):
    tt = t_ref[0][:, :C]
    z = (1.0 - ALPHA) * di_ref[...] * tt + ALPHA * h_ref[...]
    z_ref[...] = z
    m = jnp.max(z, axis=1, keepdims=True)
    lse = jnp.log(jnp.sum(jnp.exp(z - m), axis=1, keepdims=True)) + m
    out_ref[...] = z - lse


def _final(t, di, h_pad):
    return pl.pallas_call(
        _final_body,
        grid=(NC, NBLK),
        in_specs=[
            pl.BlockSpec((1, ROW_BLK, CW), lambda c, i: (c, i, 0)),
            pl.BlockSpec((ROW_BLK, C), lambda c, i: (c * NBLK + i, 0)),
            pl.BlockSpec((ROW_BLK, C), lambda c, i: (c * NBLK + i, 0)),
        ],
        out_specs=[pl.BlockSpec((ROW_BLK, C), lambda c, i: (c * NBLK + i, 0))] * 2,
        out_shape=[jax.ShapeDtypeStruct((N_PAD, C), jnp.float32)] * 2,
    )(t, di, h_pad)


def kernel(x, edge_index, W1, b1, W2, b2):
    h = _mlp(x, W1, b1, W2, b2)
    h_pad = jnp.pad(h, ((0, N_PAD - N), (0, 0)))
    src = edge_index[0]
    dst = edge_index[1]
    src_p = jnp.concatenate(
        [src, jnp.zeros((E_PAD - E,), jnp.int32)]
    ).reshape(NS * NCH_TILE, CH)
    dst_pad = jnp.concatenate([dst, jnp.full((E_PAD - E,), N_PAD, jnp.int32)])
    dst_a = jnp.where(dst_pad < HALF, dst_pad, DROW)
    dst_b = jnp.where(
        (dst_pad >= HALF) & (dst_pad < N_PAD), dst_pad - HALF, DROW
    )
    dst_p = jnp.concatenate([dst_a, dst_b]).reshape(NC * NS * NCH_TILE, CH)

    ones2 = jnp.zeros((WROWS, CW), jnp.float32).at[:N_PAD, :C].set(1.0)
    tdeg = _prop(ones2, src_p, dst_p)
    w2, g, d2, di = _prep(tdeg, h_pad)
    for _ in range(K - 1):
        t = _prop(w2, src_p, dst_p)
        w2 = _combine(t, d2, g)
    t = _prop(w2, src_p, dst_p)
    out_pad, z_pad = _final(t, di, h_pad)
    return (out_pad[:N], z_pad[:N])


# speedup vs baseline: 2.9291x; 1.0147x over previous
import jax
import jax.numpy as jnp
from jax import lax
from jax.experimental import pallas as pl
from jax.experimental.pallas import tpu as pltpu
from jax.experimental.pallas import tpu_sc as plsc

N = 10000
F_IN = 256
H = 128
C = 64
K = 10
ALPHA = 0.1

NC, NS = 2, 16
N_PAD = 10240
HALF = N_PAD // NC
A_ROWS = 5248
RPT = A_ROWS // NS
DROW = HALF
WROWS = 10496
CW = 128
E = 160000
CH = 128
NCH_TILE = 80
EPT = CH * NCH_TILE
E_PAD = EPT * NS
IDX_WIN = 8
ECHUNKS = NCH_TILE // IDX_WIN

ROW_BLK = 1024
NBLK = HALF // ROW_BLK
MLP_BLK = 1000


def _mlp_body(x_ref, w1_ref, b1_ref, w2_ref, b2_ref, o_ref):
    hid = jnp.maximum(
        jnp.dot(x_ref[...], w1_ref[...], preferred_element_type=jnp.float32)
        + b1_ref[...],
        0.0,
    )
    o_ref[...] = (
        jnp.dot(hid, w2_ref[...], preferred_element_type=jnp.float32) + b2_ref[...]
    )


def _mlp(x, W1, b1, W2, b2):
    return pl.pallas_call(
        _mlp_body,
        grid=(N // MLP_BLK,),
        in_specs=[
            pl.BlockSpec((MLP_BLK, F_IN), lambda i: (i, 0)),
            pl.BlockSpec((F_IN, H), lambda i: (0, 0)),
            pl.BlockSpec((1, H), lambda i: (0, 0)),
            pl.BlockSpec((H, C), lambda i: (0, 0)),
            pl.BlockSpec((1, C), lambda i: (0, 0)),
        ],
        out_specs=pl.BlockSpec((MLP_BLK, C), lambda i: (i, 0)),
        out_shape=jax.ShapeDtypeStruct((N, C), jnp.float32),
    )(x, W1, b1.reshape(1, H), W2, b2.reshape(1, C))


_INIT_CHUNKS = ((0, 128), (128, 128), (256, 72))


def _prop_body(
    w_hbm, srcT_hbm, dstT_hbm, t_hbm, sidx, didx, rowbuf, rowbuf2, sem, sem2, sem3, sem4, acc
):
    c = lax.axis_index("c")
    s = lax.axis_index("s")
    base = s * RPT
    for off, rows in _INIT_CHUNKS:
        pltpu.sync_copy(
            w_hbm.at[pl.ds(c * HALF + base + off, rows)], rowbuf.at[pl.ds(0, rows)]
        )
        pltpu.sync_copy(rowbuf.at[pl.ds(0, rows)], acc.at[pl.ds(base + off, rows)])
    plsc.subcore_barrier()

    for jo in range(ECHUNKS):
        pltpu.sync_copy(srcT_hbm.at[pl.ds(s * NCH_TILE + jo * IDX_WIN, IDX_WIN)], sidx)
        pltpu.sync_copy(
            dstT_hbm.at[pl.ds((c * NS + s) * NCH_TILE + jo * IDX_WIN, IDX_WIN)], didx
        )

        @pl.loop(0, IDX_WIN // 2)
        def inner(jp):
            ga = pltpu.async_copy(w_hbm.at[sidx.at[2 * jp]], rowbuf, sem)
            gb = pltpu.async_copy(w_hbm.at[sidx.at[2 * jp + 1]], rowbuf2, sem2)
            ga.wait()
            sa = pltpu.async_copy(rowbuf, acc.at[didx.at[2 * jp]], sem3, add=True)
            gb.wait()
            sb = pltpu.async_copy(rowbuf2, acc.at[didx.at[2 * jp + 1]], sem4, add=True)
            sa.wait()
            sb.wait()

    plsc.subcore_barrier()
    for off, rows in _INIT_CHUNKS:
        pltpu.sync_copy(acc.at[pl.ds(base + off, rows)], rowbuf.at[pl.ds(0, rows)])
        pltpu.sync_copy(rowbuf.at[pl.ds(0, rows)], t_hbm.at[c, pl.ds(base + off, rows)])


_prop = pl.kernel(
    _prop_body,
    out_type=jax.ShapeDtypeStruct((NC, A_ROWS, CW), jnp.float32),
    mesh=plsc.VectorSubcoreMesh(core_axis_name="c", subcore_axis_name="s"),
    scratch_types=[
        pltpu.VMEM((IDX_WIN, CH), jnp.int32),
        pltpu.VMEM((IDX_WIN, CH), jnp.int32),
        pltpu.VMEM((CH, CW), jnp.float32),
        pltpu.VMEM((CH, CW), jnp.float32),
        pltpu.SemaphoreType.DMA,
        pltpu.SemaphoreType.DMA,
        pltpu.SemaphoreType.DMA,
        pltpu.SemaphoreType.DMA,
        pltpu.VMEM_SHARED((A_ROWS, CW), jnp.float32),
    ],
)


def _prep_body(t_ref, h_ref, w2_ref, g_ref, d2_ref, di_ref):
    tt = t_ref[0]
    deg = tt[:, 0:1]
    dinv = jnp.where(deg > 0.0, lax.rsqrt(deg), 0.0)
    w0 = dinv * h_ref[...]
    w2_ref[...] = jnp.pad(w0, ((0, 0), (0, CW - C)))
    g_ref[...] = ALPHA * w0
    d2_ref[...] = jnp.broadcast_to(dinv * dinv, (ROW_BLK, C))
    di_ref[...] = jnp.broadcast_to(dinv, (ROW_BLK, C))


def _prep(t, h_pad):
    return pl.pallas_call(
        _prep_body,
        grid=(NC, NBLK),
        in_specs=[
            pl.BlockSpec((1, ROW_BLK, CW), lambda c, i: (c, i, 0)),
            pl.BlockSpec((ROW_BLK, C), lambda c, i: (c * NBLK + i, 0)),
        ],
        out_specs=[
            pl.BlockSpec((ROW_BLK, CW), lambda c, i: (c * NBLK + i, 0)),
            pl.BlockSpec((ROW_BLK, C), lambda c, i: (c * NBLK + i, 0)),
            pl.BlockSpec((ROW_BLK, C), lambda c, i: (c * NBLK + i, 0)),
            pl.BlockSpec((ROW_BLK, C), lambda c, i: (c * NBLK + i, 0)),
        ],
        out_shape=[
            jax.ShapeDtypeStruct((WROWS, CW), jnp.float32),
            jax.ShapeDtypeStruct((N_PAD, C), jnp.float32),
            jax.ShapeDtypeStruct((N_PAD, C), jnp.float32),
            jax.ShapeDtypeStruct((N_PAD, C), jnp.float32),
        ],
    )(t, h_pad)


def _combine_body(t_ref, d2_ref, g_ref, o_ref):
    tt = t_ref[0][:, :C]
    wn = (1.0 - ALPHA) * d2_ref[...] * tt + g_ref[...]
    o_ref[...] = jnp.pad(wn, ((0, 0), (0, CW - C)))


def _combine(t, d2, g):
    return pl.pallas_call(
        _combine_body,
        grid=(NC, NBLK),
        in_specs=[
            pl.BlockSpec((1, ROW_BLK, CW), lambda c, i: (c, i, 0)),
            pl.BlockSpec((ROW_BLK, C), lambda c, i: (c * NBLK + i, 0)),
            pl.BlockSpec((ROW_BLK, C), lambda c, i: (c * NBLK + i, 0)),
        ],
        out_specs=pl.BlockSpec((ROW_BLK, CW), lambda c, i: (c * NBLK + i, 0)),
        out_shape=jax.ShapeDtypeStruct((WROWS, CW), jnp.float32),
    )(t, d2, g)


def _final_body(t_ref, di_ref, h_ref, out_ref, z_ref):
    tt = t_ref[0][:, :C]
    z = (1.0 - ALPHA) * di_ref[...] * tt + ALPHA * h_ref[...]
    z_ref[...] = z
    m = jnp.max(z, axis=1, keepdims=True)
    lse = jnp.log(jnp.sum(jnp.exp(z - m), axis=1, keepdims=True)) + m
    out_ref[...] = z - lse


def _final(t, di, h_pad):
    return pl.pallas_call(
        _final_body,
        grid=(NC, NBLK),
        in_specs=[
            pl.BlockSpec((1, ROW_BLK, CW), lambda c, i: (c, i, 0)),
            pl.BlockSpec((ROW_BLK, C), lambda c, i: (c * NBLK + i, 0)),
            pl.BlockSpec((ROW_BLK, C), lambda c, i: (c * NBLK + i, 0)),
        ],
        out_specs=[pl.BlockSpec((ROW_BLK, C), lambda c, i: (c * NBLK + i, 0))] * 2,
        out_shape=[jax.ShapeDtypeStruct((N_PAD, C), jnp.float32)] * 2,
    )(t, di, h_pad)


def kernel(x, edge_index, W1, b1, W2, b2):
    h = _mlp(x, W1, b1, W2, b2)
    h_pad = jnp.pad(h, ((0, N_PAD - N), (0, 0)))
    src = edge_index[0]
    dst = edge_index[1]
    src_p = jnp.concatenate(
        [src, jnp.zeros((E_PAD - E,), jnp.int32)]
    ).reshape(NS * NCH_TILE, CH)
    dst_pad = jnp.concatenate([dst, jnp.full((E_PAD - E,), N_PAD, jnp.int32)])
    dst_a = jnp.where(dst_pad < HALF, dst_pad, DROW)
    dst_b = jnp.where(
        (dst_pad >= HALF) & (dst_pad < N_PAD), dst_pad - HALF, DROW
    )
    dst_p = jnp.concatenate([dst_a, dst_b]).reshape(NC * NS * NCH_TILE, CH)

    ones2 = jnp.zeros((WROWS, CW), jnp.float32).at[:N_PAD, :C].set(1.0)
    tdeg = _prop(ones2, src_p, dst_p)
    w2, g, d2, di = _prep(tdeg, h_pad)
    for _ in range(K - 1):
        t = _prop(w2, src_p, dst_p)
        w2 = _combine(t, d2, g)
    t = _prop(w2, src_p, dst_p)
    out_pad, z_pad = _final(t, di, h_pad)
    return (out_pad[:N], z_pad[:N])


# speedup vs baseline: 4.9406x; 1.6867x over previous
import jax
import jax.numpy as jnp
from jax import lax
from jax.experimental import pallas as pl
from jax.experimental.pallas import tpu as pltpu
from jax.experimental.pallas import tpu_sc as plsc

N = 10000
F_IN = 256
H = 128
C = 64
K = 10
ALPHA = 0.1

NC, NS = 2, 16
N_PAD = 10240
HALF = N_PAD // NC
A_ROWS = 5248
RPT = A_ROWS // NS
DROW = HALF
WROWS = 10496
CW = 64
E = 160000
CH = 128
NCH_TILE = 80
EPT = CH * NCH_TILE
E_PAD = EPT * NS
IDX_WIN = 8
ECHUNKS = NCH_TILE // IDX_WIN

ROW_BLK = 1024
NBLK = HALF // ROW_BLK
MLP_BLK = 1000


def _mlp_body(x_ref, w1_ref, b1_ref, w2_ref, b2_ref, o_ref):
    hid = jnp.maximum(
        jnp.dot(x_ref[...], w1_ref[...], preferred_element_type=jnp.float32)
        + b1_ref[...],
        0.0,
    )
    o_ref[...] = (
        jnp.dot(hid, w2_ref[...], preferred_element_type=jnp.float32) + b2_ref[...]
    )


def _mlp(x, W1, b1, W2, b2):
    return pl.pallas_call(
        _mlp_body,
        grid=(N // MLP_BLK,),
        in_specs=[
            pl.BlockSpec((MLP_BLK, F_IN), lambda i: (i, 0)),
            pl.BlockSpec((F_IN, H), lambda i: (0, 0)),
            pl.BlockSpec((1, H), lambda i: (0, 0)),
            pl.BlockSpec((H, C), lambda i: (0, 0)),
            pl.BlockSpec((1, C), lambda i: (0, 0)),
        ],
        out_specs=pl.BlockSpec((MLP_BLK, C), lambda i: (i, 0)),
        out_shape=jax.ShapeDtypeStruct((N, C), jnp.float32),
    )(x, W1, b1.reshape(1, H), W2, b2.reshape(1, C))


_INIT_CHUNKS = ((0, 128), (128, 128), (256, 72))


def _prop_body(
    w_hbm, srcT_hbm, dstT_hbm, t_hbm, sidx, didx, rowbuf, rowbuf2, sem, sem2, sem3, sem4, acc
):
    c = lax.axis_index("c")
    s = lax.axis_index("s")
    base = s * RPT
    for off, rows in _INIT_CHUNKS:
        pltpu.sync_copy(
            w_hbm.at[pl.ds(c * HALF + base + off, rows)], rowbuf.at[pl.ds(0, rows)]
        )
        pltpu.sync_copy(rowbuf.at[pl.ds(0, rows)], acc.at[pl.ds(base + off, rows)])
    plsc.subcore_barrier()

    for jo in range(ECHUNKS):
        pltpu.sync_copy(srcT_hbm.at[pl.ds(s * NCH_TILE + jo * IDX_WIN, IDX_WIN)], sidx)
        pltpu.sync_copy(
            dstT_hbm.at[pl.ds((c * NS + s) * NCH_TILE + jo * IDX_WIN, IDX_WIN)], didx
        )

        @pl.loop(0, IDX_WIN // 2)
        def inner(jp):
            ga = pltpu.async_copy(w_hbm.at[sidx.at[2 * jp]], rowbuf, sem)
            gb = pltpu.async_copy(w_hbm.at[sidx.at[2 * jp + 1]], rowbuf2, sem2)
            ga.wait()
            sa = pltpu.async_copy(rowbuf, acc.at[didx.at[2 * jp]], sem3, add=True)
            gb.wait()
            sb = pltpu.async_copy(rowbuf2, acc.at[didx.at[2 * jp + 1]], sem4, add=True)
            sa.wait()
            sb.wait()

    plsc.subcore_barrier()
    for off, rows in _INIT_CHUNKS:
        pltpu.sync_copy(acc.at[pl.ds(base + off, rows)], rowbuf.at[pl.ds(0, rows)])
        pltpu.sync_copy(rowbuf.at[pl.ds(0, rows)], t_hbm.at[c, pl.ds(base + off, rows)])


_prop = pl.kernel(
    _prop_body,
    out_type=jax.ShapeDtypeStruct((NC, A_ROWS, CW), jnp.float32),
    mesh=plsc.VectorSubcoreMesh(core_axis_name="c", subcore_axis_name="s"),
    compiler_params=pltpu.CompilerParams(use_tc_tiling_on_sc=False),
    scratch_types=[
        pltpu.VMEM((IDX_WIN, CH), jnp.int32),
        pltpu.VMEM((IDX_WIN, CH), jnp.int32),
        pltpu.VMEM((CH, CW), jnp.float32),
        pltpu.VMEM((CH, CW), jnp.float32),
        pltpu.SemaphoreType.DMA,
        pltpu.SemaphoreType.DMA,
        pltpu.SemaphoreType.DMA,
        pltpu.SemaphoreType.DMA,
        pltpu.VMEM_SHARED((A_ROWS, CW), jnp.float32),
    ],
)


def _prep_body(t_ref, h_ref, w2_ref, g_ref, d2_ref, di_ref):
    tt = t_ref[0]
    deg = tt[:, 0:1]
    dinv = jnp.where(deg > 0.0, lax.rsqrt(deg), 0.0)
    w0 = dinv * h_ref[...]
    w2_ref[...] = jnp.pad(w0, ((0, 0), (0, CW - C)))
    g_ref[...] = ALPHA * w0
    d2_ref[...] = jnp.broadcast_to(dinv * dinv, (ROW_BLK, C))
    di_ref[...] = jnp.broadcast_to(dinv, (ROW_BLK, C))


def _prep(t, h_pad):
    return pl.pallas_call(
        _prep_body,
        grid=(NC, NBLK),
        in_specs=[
            pl.BlockSpec((1, ROW_BLK, CW), lambda c, i: (c, i, 0)),
            pl.BlockSpec((ROW_BLK, C), lambda c, i: (c * NBLK + i, 0)),
        ],
        out_specs=[
            pl.BlockSpec((ROW_BLK, CW), lambda c, i: (c * NBLK + i, 0)),
            pl.BlockSpec((ROW_BLK, C), lambda c, i: (c * NBLK + i, 0)),
            pl.BlockSpec((ROW_BLK, C), lambda c, i: (c * NBLK + i, 0)),
            pl.BlockSpec((ROW_BLK, C), lambda c, i: (c * NBLK + i, 0)),
        ],
        out_shape=[
            jax.ShapeDtypeStruct((WROWS, CW), jnp.float32),
            jax.ShapeDtypeStruct((N_PAD, C), jnp.float32),
            jax.ShapeDtypeStruct((N_PAD, C), jnp.float32),
            jax.ShapeDtypeStruct((N_PAD, C), jnp.float32),
        ],
    )(t, h_pad)


def _combine_body(t_ref, d2_ref, g_ref, o_ref):
    tt = t_ref[0][:, :C]
    wn = (1.0 - ALPHA) * d2_ref[...] * tt + g_ref[...]
    o_ref[...] = jnp.pad(wn, ((0, 0), (0, CW - C)))


def _combine(t, d2, g):
    return pl.pallas_call(
        _combine_body,
        grid=(NC, NBLK),
        in_specs=[
            pl.BlockSpec((1, ROW_BLK, CW), lambda c, i: (c, i, 0)),
            pl.BlockSpec((ROW_BLK, C), lambda c, i: (c * NBLK + i, 0)),
            pl.BlockSpec((ROW_BLK, C), lambda c, i: (c * NBLK + i, 0)),
        ],
        out_specs=pl.BlockSpec((ROW_BLK, CW), lambda c, i: (c * NBLK + i, 0)),
        out_shape=jax.ShapeDtypeStruct((WROWS, CW), jnp.float32),
    )(t, d2, g)


def _final_body(t_ref, di_ref, h_ref, out_ref, z_ref):
    tt = t_ref[0][:, :C]
    z = (1.0 - ALPHA) * di_ref[...] * tt + ALPHA * h_ref[...]
    z_ref[...] = z
    m = jnp.max(z, axis=1, keepdims=True)
    lse = jnp.log(jnp.sum(jnp.exp(z - m), axis=1, keepdims=True)) + m
    out_ref[...] = z - lse


def _final(t, di, h_pad):
    return pl.pallas_call(
        _final_body,
        grid=(NC, NBLK),
        in_specs=[
            pl.BlockSpec((1, ROW_BLK, CW), lambda c, i: (c, i, 0)),
            pl.BlockSpec((ROW_BLK, C), lambda c, i: (c * NBLK + i, 0)),
            pl.BlockSpec((ROW_BLK, C), lambda c, i: (c * NBLK + i, 0)),
        ],
        out_specs=[pl.BlockSpec((ROW_BLK, C), lambda c, i: (c * NBLK + i, 0))] * 2,
        out_shape=[jax.ShapeDtypeStruct((N_PAD, C), jnp.float32)] * 2,
    )(t, di, h_pad)


def kernel(x, edge_index, W1, b1, W2, b2):
    h = _mlp(x, W1, b1, W2, b2)
    h_pad = jnp.pad(h, ((0, N_PAD - N), (0, 0)))
    src = edge_index[0]
    dst = edge_index[1]
    src_p = jnp.concatenate(
        [src, jnp.zeros((E_PAD - E,), jnp.int32)]
    ).reshape(NS * NCH_TILE, CH)
    dst_pad = jnp.concatenate([dst, jnp.full((E_PAD - E,), N_PAD, jnp.int32)])
    dst_a = jnp.where(dst_pad < HALF, dst_pad, DROW)
    dst_b = jnp.where(
        (dst_pad >= HALF) & (dst_pad < N_PAD), dst_pad - HALF, DROW
    )
    dst_p = jnp.concatenate([dst_a, dst_b]).reshape(NC * NS * NCH_TILE, CH)

    ones2 = jnp.zeros((WROWS, CW), jnp.float32).at[:N_PAD, :C].set(1.0)
    tdeg = _prop(ones2, src_p, dst_p)
    w2, g, d2, di = _prep(tdeg, h_pad)
    for _ in range(K - 1):
        t = _prop(w2, src_p, dst_p)
        w2 = _combine(t, d2, g)
    t = _prop(w2, src_p, dst_p)
    out_pad, z_pad = _final(t, di, h_pad)
    return (out_pad[:N], z_pad[:N])
